# baseline (speedup 1.0000x reference)
import functools

import jax
import jax.numpy as jnp
from jax import lax
from jax.experimental import pallas as pl
from jax.experimental.pallas import tpu as pltpu
from jax.experimental.pallas import tpu_sc as plsc

DIM = 64
NC, NS = 2, 16
NW = NC * NS
CHUNK = 100
HALF = CHUNK // 2
NBUF = 8
K = 4
VPAD = 1024


@functools.cache
def _emb_call(batch, seq):
    n_total = batch * seq
    n_per_w = n_total // NW
    n_chunks = n_per_w // CHUNK
    n_rows = (n_total * DIM) // 128
    rows_per_chunk = (CHUNK * DIM) // 128
    n_outer = n_chunks // NBUF
    assert seq == 2 * CHUNK and n_chunks % NBUF == 0 and n_outer >= 2
    mesh = plsc.VectorSubcoreMesh(core_axis_name="c", subcore_axis_name="s")

    @functools.partial(
        pl.kernel,
        out_type=jax.ShapeDtypeStruct((n_rows, 128), jnp.float32),
        mesh=mesh,
        scratch_types=(
            [pltpu.VMEM((2 * n_chunks, HALF), jnp.int32),
             pltpu.VMEM((2, NBUF, HALF, DIM), jnp.float32),
             pltpu.VMEM_SHARED((VPAD, DIM), jnp.float32),
             pltpu.VMEM((16, DIM), jnp.float32)]
            + [pltpu.SemaphoreType.DMA] * (2 * NBUF)
        ),
        compiler_params=pltpu.CompilerParams(use_tc_tiling_on_sc=False),
    )
    def emb(idx_hbm, table_hbm, out_hbm, idx_v, rows_v, table_sp, peek_v,
            *sems):
        sem_g, sem_s = sems[:NBUF], sems[NBUF:]
        sid = lax.axis_index("s")
        wid = sid * NC + lax.axis_index("c")
        pltpu.sync_copy(table_hbm.at[pl.ds(sid * 64, 64)],
                        table_sp.at[pl.ds(sid * 64, 64)])
        pltpu.sync_copy(idx_hbm.at[pl.ds(wid * 2 * n_chunks, 2 * n_chunks)],
                        idx_v)
        plsc.subcore_barrier()
        nb = lax.rem(sid + 1, 16)
        pltpu.sync_copy(table_sp.at[pl.ds(nb * 64 + 48, 16)], peek_v)
        plsc.subcore_barrier()
        base_r = wid * n_chunks * rows_per_chunk

        def gather(j, b):
            pltpu.async_copy(table_sp.at[idx_v.at[2 * j]],
                             rows_v.at[0, b], sem_g[b])
            pltpu.async_copy(table_sp.at[idx_v.at[2 * j + 1]],
                             rows_v.at[1, b], sem_g[b])

        def gather_wait(j, b):
            pltpu.make_async_copy(table_sp.at[idx_v.at[2 * j]],
                                  rows_v.at[0, b], sem_g[b]).wait()
            pltpu.make_async_copy(table_sp.at[idx_v.at[2 * j + 1]],
                                  rows_v.at[1, b], sem_g[b]).wait()

        def _out_half(j, p):
            return out_hbm.at[pl.ds(base_r + j * rows_per_chunk,
                                    rows_per_chunk),
                              pl.ds(p * DIM, DIM)]

        def store(j, b):
            pltpu.async_copy(rows_v.at[0, b], _out_half(j, 0), sem_s[b])
            pltpu.async_copy(rows_v.at[1, b], _out_half(j, 1), sem_s[b])

        def store_wait(j, b):
            pltpu.make_async_copy(rows_v.at[0, b], _out_half(j, 0),
                                  sem_s[b]).wait()
            pltpu.make_async_copy(rows_v.at[1, b], _out_half(j, 1),
                                  sem_s[b]).wait()

        for b in range(K):
            gather(b, b)

        for b in range(NBUF):
            bb = (b + K) % NBUF
            if b >= NBUF - K:
                store_wait(b - (NBUF - K), bb)
            gather(b + K, bb)
            gather_wait(b, b)
            store(b, b)

        def outer(o, carry):
            j0 = o * NBUF
            for b in range(NBUF):
                j = j0 + b
                bb = (b + K) % NBUF
                store_wait(j - (NBUF - K), bb)
                gather(j + K, bb)
                gather_wait(j, b)
                store(j, b)
            return carry

        lax.fori_loop(1, n_outer - 1, outer, 0)

        j0 = (n_outer - 1) * NBUF
        for b in range(NBUF):
            j = j0 + b
            bb = (b + K) % NBUF
            if b < NBUF - K:
                store_wait(j - (NBUF - K), bb)
                gather(j + K, bb)
            gather_wait(j, b)
            store(j, b)

        for b in range(NBUF):
            store_wait(j0 + b, b)

    return emb


def kernel(x, table):
    b, s = x.shape
    idx = (x.reshape(b, 2, HALF, 2)
             .transpose(0, 1, 3, 2)
             .reshape(b * 4, HALF)
             .astype(jnp.int32))
    tab = jnp.pad(table.astype(jnp.float32), ((0, VPAD - table.shape[0]),
                                              (0, 0)))
    out = _emb_call(b, s)(idx, tab)
    return out.reshape(b, s, DIM)


# speedup vs baseline: 1.3990x; 1.3990x over previous
import functools

import jax
import jax.numpy as jnp
from jax import lax
from jax.experimental import pallas as pl
from jax.experimental.pallas import tpu as pltpu
from jax.experimental.pallas import tpu_sc as plsc

DIM = 64
NC, NS = 2, 16
NW = NC * NS
CHUNK = 200
HALF = CHUNK // 2
NBUF = 2
K = 1
VPAD = 1024


@functools.cache
def _emb_call(batch, seq):
    n_total = batch * seq
    b_per_w = batch // NW
    n_chunks = b_per_w
    n_outer = n_chunks // NBUF
    assert seq == CHUNK and n_chunks % NBUF == 0 and n_outer >= 2
    mesh = plsc.VectorSubcoreMesh(core_axis_name="c", subcore_axis_name="s")

    @functools.partial(
        pl.kernel,
        out_type=jax.ShapeDtypeStruct((batch, seq, DIM), jnp.float32),
        mesh=mesh,
        scratch_types=(
            [pltpu.VMEM((2 * n_chunks, HALF), jnp.int32),
             pltpu.VMEM((NBUF, CHUNK, DIM), jnp.float32),
             pltpu.VMEM_SHARED((VPAD, DIM), jnp.float32),
             pltpu.VMEM((16, DIM), jnp.float32)]
            + [pltpu.SemaphoreType.DMA] * (2 * NBUF)
        ),
        compiler_params=pltpu.CompilerParams(use_tc_tiling_on_sc=True),
    )
    def emb(idx_hbm, table_hbm, out_hbm, idx_v, rows_v, table_sp, peek_v,
            *sems):
        sem_g, sem_s = sems[:NBUF], sems[NBUF:]
        sid = lax.axis_index("s")
        wid = sid * NC + lax.axis_index("c")
        pltpu.sync_copy(table_hbm.at[pl.ds(sid * 64, 64)],
                        table_sp.at[pl.ds(sid * 64, 64)])
        pltpu.sync_copy(idx_hbm.at[pl.ds(wid * 2 * n_chunks, 2 * n_chunks)],
                        idx_v)
        plsc.subcore_barrier()
        nb = lax.rem(sid + 1, 16)
        pltpu.sync_copy(table_sp.at[pl.ds(nb * 64 + 48, 16)], peek_v)
        plsc.subcore_barrier()
        base_b = wid * b_per_w

        def gather(j, b):
            pltpu.async_copy(table_sp.at[idx_v.at[2 * j]],
                             rows_v.at[b, pl.ds(0, HALF)], sem_g[b])
            pltpu.async_copy(table_sp.at[idx_v.at[2 * j + 1]],
                             rows_v.at[b, pl.ds(HALF, HALF)], sem_g[b])

        def gather_wait(j, b):
            pltpu.make_async_copy(table_sp.at[idx_v.at[2 * j]],
                                  rows_v.at[b, pl.ds(0, HALF)],
                                  sem_g[b]).wait()
            pltpu.make_async_copy(table_sp.at[idx_v.at[2 * j + 1]],
                                  rows_v.at[b, pl.ds(HALF, HALF)],
                                  sem_g[b]).wait()

        def store(j, b):
            pltpu.async_copy(rows_v.at[b], out_hbm.at[base_b + j], sem_s[b])

        def store_wait(j, b):
            pltpu.make_async_copy(rows_v.at[b], out_hbm.at[base_b + j],
                                  sem_s[b]).wait()

        for b in range(K):
            gather(b, b)

        for b in range(NBUF):
            bb = (b + K) % NBUF
            if b >= NBUF - K:
                store_wait(b - (NBUF - K), bb)
            gather(b + K, bb)
            gather_wait(b, b)
            store(b, b)

        def outer(o, carry):
            j0 = o * NBUF
            for b in range(NBUF):
                j = j0 + b
                bb = (b + K) % NBUF
                store_wait(j - (NBUF - K), bb)
                gather(j + K, bb)
                gather_wait(j, b)
                store(j, b)
            return carry

        lax.fori_loop(1, n_outer - 1, outer, 0)

        j0 = (n_outer - 1) * NBUF
        for b in range(NBUF):
            j = j0 + b
            bb = (b + K) % NBUF
            if b < NBUF - K:
                store_wait(j - (NBUF - K), bb)
                gather(j + K, bb)
            gather_wait(j, b)
            store(j, b)

        for b in range(NBUF):
            store_wait(j0 + b, b)

    return emb


def kernel(x, table):
    b, s = x.shape
    idx = x.reshape(2 * b, HALF).astype(jnp.int32)
    tab = jnp.pad(table.astype(jnp.float32),
                  ((0, VPAD - table.shape[0]), (0, 0)))
    return _emb_call(b, s)(idx, tab)
